# Initial kernel scaffold; baseline (speedup 1.0000x reference)
import jax
import jax.numpy as jnp
from jax.experimental import pallas as pl


def kernel(x, edge_index, W1, a_src1, a_dst1, b1, W2, a_src2, a_dst2, b2):
    raise NotImplementedError("write your pallas kernel here")


# speedup vs baseline: 1.6061x; 1.6061x over previous
import functools

import jax
import jax.numpy as jnp
from jax.experimental import pallas as pl

_N = 10000
_ROW_BLK = 1000


def _mm_body(x_ref, w_ref, o_ref):
    o_ref[...] = jnp.dot(x_ref[...], w_ref[...],
                         preferred_element_type=jnp.float32)


def _matmul(x, w):
    n, k = x.shape
    k2, m = w.shape
    grid = (n // _ROW_BLK,)
    return pl.pallas_call(
        _mm_body,
        grid=grid,
        in_specs=[
            pl.BlockSpec((_ROW_BLK, k), lambda i: (i, 0)),
            pl.BlockSpec((k, m), lambda i: (0, 0)),
        ],
        out_specs=pl.BlockSpec((_ROW_BLK, m), lambda i: (i, 0)),
        out_shape=jax.ShapeDtypeStruct((n, m), jnp.float32),
    )(x, w)


def _gat_layer(x, src, dst, W, a_src, a_dst, b, apply_relu):
    n = x.shape[0]
    h = _matmul(x, W)
    asrc = h @ a_src[0]
    adst = h @ a_dst[0]
    e = asrc[src] + adst[dst]
    e = jnp.where(e >= 0, e, 0.2 * e)
    w = jnp.exp(e)
    U = jax.ops.segment_sum(h[src] * w[:, None], dst, num_segments=n)
    den = jax.ops.segment_sum(w, dst, num_segments=n)
    e_self = asrc + adst
    w_self = jnp.exp(jnp.where(e_self >= 0, e_self, 0.2 * e_self))
    out = (U + w_self[:, None] * h) / (den + w_self)[:, None] + b
    if apply_relu:
        out = jnp.maximum(out, 0.0)
    return out


def kernel(x, edge_index, W1, a_src1, a_dst1, b1, W2, a_src2, a_dst2, b2):
    src, dst = edge_index[0], edge_index[1]
    h = _gat_layer(x, src, dst, W1, a_src1, a_dst1, b1, True)
    out = _gat_layer(h, src, dst, W2, a_src2, a_dst2, b2, False)
    return out
